# baseline (device time: 29628 ns/iter reference)
import jax
import jax.numpy as jnp
from jax import lax
from jax.experimental import pallas as pl
from jax.experimental.pallas import tpu as pltpu

N_DEV = 4
N_LAYERS = 3


def kernel(x, Win0, Wout0, Win1, Wout1, Win2, Wout2):
    B, D = x.shape
    H = Win0.shape[1]
    R = B // N_DEV

    def body(x_ref, win0_ref, wout0_ref, win1_ref, wout1_ref, win2_ref,
             wout2_ref, out_ref, bc_ref, part_ref, rs_ref, win_v, wout_v,
             send_b, recv_b, send_rs, recv_rs, load_sems):
        my = lax.axis_index("i")

        loads = {}
        for l, (wi, wo) in ((1, (win1_ref, wout1_ref)),
                            (2, (win2_ref, wout2_ref))):
            i = l - 1
            cpi = pltpu.make_async_copy(wi, win_v.at[i], load_sems.at[i, 0])
            cpo = pltpu.make_async_copy(wo, wout_v.at[i], load_sems.at[i, 1])
            cpi.start()
            cpo.start()
            loads[l] = (cpi, cpo)
        wins = [win0_ref, win_v.at[0], win_v.at[1]]
        wouts = [wout0_ref, wout_v.at[0], wout_v.at[1]]

        started = []

        def mlp_chunk(xc, l):
            h = jnp.maximum(
                jnp.dot(xc, wins[l][:, :],
                        preferred_element_type=jnp.float32),
                0.0)
            return jnp.dot(h, wouts[l][:, :],
                           preferred_element_type=jnp.float32)

        def bcast_chunk(l, c):
            src = bc_ref.at[l, my, pl.ds(c * R, R), :]
            for o in (2, 1, 3):
                e = (my + o) % N_DEV
                rdma = pltpu.make_async_remote_copy(
                    src_ref=src, dst_ref=src,
                    send_sem=send_b.at[l, c, o - 1],
                    recv_sem=recv_b.at[l, my, c],
                    device_id=(e,), device_id_type=pl.DeviceIdType.MESH,
                )
                rdma.start()
                started.append(rdma)

        def gather_chunk(l, c):
            acc = bc_ref[l, my, pl.ds(c * R, R), :].astype(jnp.float32)
            for o in (1, 3, 2):
                s = (my + o) % N_DEV
                pltpu.make_async_remote_copy(
                    src_ref=bc_ref.at[l, s, pl.ds(c * R, R), :],
                    dst_ref=bc_ref.at[l, s, pl.ds(c * R, R), :],
                    send_sem=send_b.at[l, c, 0],
                    recv_sem=recv_b.at[l, s, c],
                    device_id=(s,), device_id_type=pl.DeviceIdType.MESH,
                ).wait_recv()
                acc = acc + bc_ref[l, s, pl.ds(c * R, R), :].astype(jnp.float32)
            return acc

        def rs_send(c):
            return pltpu.make_async_remote_copy(
                src_ref=part_ref.at[c],
                dst_ref=rs_ref.at[my],
                send_sem=send_rs.at[c],
                recv_sem=recv_rs.at[my],
                device_id=(c,), device_id_type=pl.DeviceIdType.MESH,
            )

        for l in (0, 1):
            if l > 0:
                for cp in loads[l]:
                    cp.wait()
            for c in range(N_DEV):
                if l == 0:
                    xc = x_ref[pl.ds(c * R, R), :]
                else:
                    xc = gather_chunk(0, c)
                bc_ref[l, my, pl.ds(c * R, R), :] = \
                    mlp_chunk(xc, l).astype(jnp.bfloat16)
                bcast_chunk(l, c)

        for cp in loads[2]:
            cp.wait()
        for c in range(N_DEV):
            xc = gather_chunk(1, c)
            part_ref[c, :, :] = mlp_chunk(xc, 2).astype(jnp.bfloat16)

            @pl.when(c != my)
            def _():
                rs_send(c).start()

        acc = part_ref[my, :, :].astype(jnp.float32)
        for o in (1, 3, 2):
            s = (my + o) % N_DEV
            pltpu.make_async_remote_copy(
                src_ref=rs_ref.at[s], dst_ref=rs_ref.at[s],
                send_sem=send_rs.at[0], recv_sem=recv_rs.at[s],
                device_id=(s,), device_id_type=pl.DeviceIdType.MESH,
            ).wait_recv()
            acc = acc + rs_ref[s, :, :].astype(jnp.float32)
        out_ref[:, :] = acc

        for rdma in started:
            rdma.wait_send()
        for c in range(N_DEV):
            @pl.when(c != my)
            def _():
                rs_send(c).wait_send()

    return pl.pallas_call(
        body,
        out_shape=jax.ShapeDtypeStruct((R, D), jnp.float32),
        in_specs=[pl.BlockSpec(memory_space=pltpu.VMEM)] * 3
        + [pl.BlockSpec(memory_space=pl.ANY)] * 4,
        out_specs=pl.BlockSpec(memory_space=pltpu.VMEM),
        scratch_shapes=[
            pltpu.VMEM((2, N_DEV, B, D), jnp.bfloat16),
            pltpu.VMEM((N_DEV, R, D), jnp.bfloat16),
            pltpu.VMEM((N_DEV, R, D), jnp.bfloat16),
            pltpu.VMEM((2, D, H), jnp.float32),
            pltpu.VMEM((2, H, D), jnp.float32),
            pltpu.SemaphoreType.DMA((2, N_DEV, N_DEV - 1)),
            pltpu.SemaphoreType.DMA((2, N_DEV, N_DEV)),
            pltpu.SemaphoreType.DMA((N_DEV,)),
            pltpu.SemaphoreType.DMA((N_DEV,)),
            pltpu.SemaphoreType.DMA((2, 2)),
        ],
    )(x, Win0, Wout0, Win1, Wout1, Win2, Wout2)
